# Initial kernel scaffold; baseline (speedup 1.0000x reference)
import functools

import jax
import jax.numpy as jnp
from jax import lax
from jax.experimental import pallas as pl
from jax.experimental.pallas import tpu as pltpu
from jax.experimental.pallas import tpu_sc as plsc

NSEG = 512
NROW = 320000
NCOL = 128
NC = 2
NS = 16
CHALF = NCOL // NC
RPT = NROW // NS
BLK = 125
NB = RPT // BLK
SEG_PT = NSEG // NS


def _body(x_hbm, ids_hbm, out_hbm,
          ids_v, xb0, xb1, ones_v, zb, sbuf, cbuf, rbuf, obuf,
          table_sh, counts_sh, sem0, sem1, sem2):
    c = lax.axis_index("c")
    s = lax.axis_index("s")
    col0 = c * CHALF
    row0 = s * RPT
    seg0 = s * SEG_PT

    pltpu.sync_copy(ids_hbm.at[s], ids_v)

    one16 = jnp.ones((16,), jnp.float32)
    zero16 = jnp.zeros((16,), jnp.float32)
    for i in range(8):
        ones_v[pl.ds(i * 16, 16)] = one16
    for i in range(2):
        cbuf[pl.ds(i * 16, 16)] = zero16
    for r in range(SEG_PT):
        for j in range(CHALF // 16):
            zb[r, pl.ds(j * 16, 16)] = zero16

    pltpu.sync_copy(zb, table_sh.at[pl.ds(seg0, SEG_PT), :])
    pltpu.sync_copy(cbuf, counts_sh.at[pl.ds(seg0, SEG_PT)])
    plsc.subcore_barrier()

    def start_load(b_idx, buf, sem):
        pltpu.async_copy(
            x_hbm.at[pl.ds(row0 + b_idx * BLK, BLK), pl.ds(col0, CHALF)],
            buf, sem)

    def wait_load(buf, sem):
        pltpu.make_async_copy(
            x_hbm.at[pl.ds(0, BLK), pl.ds(0, CHALF)], buf, sem).wait()

    def consume(b_idx, buf):
        idx = ids_v.at[b_idx]
        cnt_cp = pltpu.async_copy(
            ones_v.at[pl.ds(0, BLK)], counts_sh.at[idx], sem2, add=True)
        pltpu.sync_copy(buf, table_sh.at[idx], add=True)
        cnt_cp.wait()

    start_load(0, xb0, sem0)

    def pair(i, carry):
        g = i * 2
        start_load(g + 1, xb1, sem1)
        wait_load(xb0, sem0)
        consume(g, xb0)
        start_load(g + 2, xb0, sem0)
        wait_load(xb1, sem1)
        consume(g + 1, xb1)
        return carry

    lax.fori_loop(0, NB // 2 - 1, pair, 0)

    start_load(NB - 1, xb1, sem1)
    wait_load(xb0, sem0)
    consume(NB - 2, xb0)
    wait_load(xb1, sem1)
    consume(NB - 1, xb1)

    plsc.subcore_barrier()

    pltpu.sync_copy(table_sh.at[pl.ds(seg0, SEG_PT), :], sbuf)
    pltpu.sync_copy(counts_sh.at[pl.ds(seg0, SEG_PT)], cbuf)
    for i in range(SEG_PT // 16):
        cv = cbuf[pl.ds(i * 16, 16)]
        rbuf[pl.ds(i * 16, 16)] = 1.0 / jnp.maximum(cv, 1.0)
    for r in range(SEG_PT):
        rv = plsc.load_gather(rbuf, [jnp.full((16,), r, jnp.int32)])
        for j in range(CHALF // 16):
            obuf[r, pl.ds(j * 16, 16)] = sbuf[r, pl.ds(j * 16, 16)] * rv
    pltpu.sync_copy(obuf, out_hbm.at[pl.ds(seg0, SEG_PT), pl.ds(col0, CHALF)])


_pool = functools.partial(
    pl.kernel,
    out_type=jax.ShapeDtypeStruct((NSEG, NCOL), jnp.float32),
    mesh=plsc.VectorSubcoreMesh(core_axis_name="c", subcore_axis_name="s"),
    scratch_types=[
        pltpu.VMEM((NB, BLK), jnp.int32),
        pltpu.VMEM((BLK, CHALF), jnp.float32),
        pltpu.VMEM((BLK, CHALF), jnp.float32),
        pltpu.VMEM((128,), jnp.float32),
        pltpu.VMEM((SEG_PT, CHALF), jnp.float32),
        pltpu.VMEM((SEG_PT, CHALF), jnp.float32),
        pltpu.VMEM((SEG_PT,), jnp.float32),
        pltpu.VMEM((SEG_PT,), jnp.float32),
        pltpu.VMEM((SEG_PT, CHALF), jnp.float32),
        pltpu.VMEM_SHARED((NSEG, CHALF), jnp.float32),
        pltpu.VMEM_SHARED((NSEG,), jnp.float32),
        pltpu.SemaphoreType.DMA,
        pltpu.SemaphoreType.DMA,
        pltpu.SemaphoreType.DMA,
    ],
)(_body)


@jax.jit
def kernel(x, batch):
    ids = batch.astype(jnp.int32).reshape(NS, NB, BLK)
    return _pool(x, ids)


# speedup vs baseline: 6.2372x; 6.2372x over previous
import functools

import jax
import jax.numpy as jnp
from jax import lax
from jax.experimental import pallas as pl
from jax.experimental.pallas import tpu as pltpu
from jax.experimental.pallas import tpu_sc as plsc

NSEG = 512
NROW = 320000
NCOL = 128
NC = 2
NS = 16
CHALF = NCOL // NC
RPT = NROW // NS
BLK = 125
NB = RPT // BLK
SEG_PT = NSEG // NS
CW = 16


def _body(x_hbm, ids_hbm, out_hbm,
          ids_v, xb0, xb1, ones_v, zb, zc, sbuf, tbuf, cbuf, ctbuf,
          obuf, table_sh, counts_sh, sem0, sem1, sem2):
    c = lax.axis_index("c")
    s = lax.axis_index("s")
    col0 = c * CHALF
    row0 = s * RPT
    seg0 = s * SEG_PT
    priv0 = s * NSEG

    pltpu.sync_copy(ids_hbm.at[s], ids_v)

    one16 = jnp.ones((16,), jnp.float32)
    zero16 = jnp.zeros((16,), jnp.float32)
    for r in range(BLK):
        ones_v[r, :] = one16
    for r in range(128):
        zc[r, :] = zero16
    for r in range(SEG_PT):
        for j in range(CHALF // 16):
            zb[r, pl.ds(j * 16, 16)] = zero16

    for k in range(NSEG // SEG_PT):
        pltpu.sync_copy(zb, table_sh.at[pl.ds(priv0 + k * SEG_PT, SEG_PT), :])
    for k in range(NSEG // 128):
        pltpu.sync_copy(zc, counts_sh.at[pl.ds(priv0 + k * 128, 128), :])
    plsc.subcore_barrier()

    def start_load(b_idx, buf, sem):
        pltpu.async_copy(
            x_hbm.at[pl.ds(row0 + b_idx * BLK, BLK), pl.ds(col0, CHALF)],
            buf, sem)

    def wait_load(buf, sem):
        pltpu.make_async_copy(
            x_hbm.at[pl.ds(0, BLK), pl.ds(0, CHALF)], buf, sem).wait()

    def consume(b_idx, buf):
        idx = ids_v.at[b_idx]
        cnt_cp = pltpu.async_copy(
            ones_v, counts_sh.at[idx], sem2, add=True)
        pltpu.sync_copy(buf, table_sh.at[idx], add=True)
        cnt_cp.wait()

    start_load(0, xb0, sem0)

    def pair(i, carry):
        g = i * 2
        start_load(g + 1, xb1, sem1)
        wait_load(xb0, sem0)
        consume(g, xb0)
        start_load(g + 2, xb0, sem0)
        wait_load(xb1, sem1)
        consume(g + 1, xb1)
        return carry

    lax.fori_loop(0, NB // 2 - 1, pair, 0)

    start_load(NB - 1, xb1, sem1)
    wait_load(xb0, sem0)
    consume(NB - 2, xb0)
    wait_load(xb1, sem1)
    consume(NB - 1, xb1)

    plsc.subcore_barrier()

    pltpu.sync_copy(table_sh.at[pl.ds(seg0, SEG_PT), :], sbuf)
    pltpu.sync_copy(counts_sh.at[pl.ds(seg0, SEG_PT), :], cbuf)

    def red(t, carry):
        base = t * NSEG + seg0
        pltpu.sync_copy(table_sh.at[pl.ds(base, SEG_PT), :], tbuf)
        pltpu.sync_copy(counts_sh.at[pl.ds(base, SEG_PT), :], ctbuf)
        for r in range(SEG_PT):
            for j in range(CHALF // 16):
                sl = pl.ds(j * 16, 16)
                sbuf[r, sl] += tbuf[r, sl]
            cbuf[r, :] += ctbuf[r, :]
        return carry

    lax.fori_loop(1, NS, red, 0)

    for r in range(SEG_PT):
        rv = 1.0 / jnp.maximum(cbuf[r, :], 1.0)
        for j in range(CHALF // 16):
            obuf[r, pl.ds(j * 16, 16)] = sbuf[r, pl.ds(j * 16, 16)] * rv
    pltpu.sync_copy(obuf, out_hbm.at[pl.ds(seg0, SEG_PT), pl.ds(col0, CHALF)])


_pool = functools.partial(
    pl.kernel,
    out_type=jax.ShapeDtypeStruct((NSEG, NCOL), jnp.float32),
    mesh=plsc.VectorSubcoreMesh(core_axis_name="c", subcore_axis_name="s",
                                num_cores=NC, num_subcores=NS),
    compiler_params=pltpu.CompilerParams(use_tc_tiling_on_sc=False,
                                         needs_layout_passes=False),
    scratch_types=[
        pltpu.VMEM((NB, BLK), jnp.int32),
        pltpu.VMEM((BLK, CHALF), jnp.float32),
        pltpu.VMEM((BLK, CHALF), jnp.float32),
        pltpu.VMEM((BLK, CW), jnp.float32),
        pltpu.VMEM((SEG_PT, CHALF), jnp.float32),
        pltpu.VMEM((128, CW), jnp.float32),
        pltpu.VMEM((SEG_PT, CHALF), jnp.float32),
        pltpu.VMEM((SEG_PT, CHALF), jnp.float32),
        pltpu.VMEM((SEG_PT, CW), jnp.float32),
        pltpu.VMEM((SEG_PT, CW), jnp.float32),
        pltpu.VMEM((SEG_PT, CHALF), jnp.float32),
        pltpu.VMEM_SHARED((NS * NSEG, CHALF), jnp.float32),
        pltpu.VMEM_SHARED((NS * NSEG, CW), jnp.float32),
        pltpu.SemaphoreType.DMA,
        pltpu.SemaphoreType.DMA,
        pltpu.SemaphoreType.DMA,
    ],
)(_body)


@jax.jit
def kernel(x, batch):
    ids = batch.astype(jnp.int32).reshape(NS, NB, BLK)
    ids = ids + (jnp.arange(NS, dtype=jnp.int32) * NSEG)[:, None, None]
    return _pool(x, ids)


# speedup vs baseline: 6.7342x; 1.0797x over previous
import functools

import jax
import jax.numpy as jnp
from jax import lax
from jax.experimental import pallas as pl
from jax.experimental.pallas import tpu as pltpu
from jax.experimental.pallas import tpu_sc as plsc

NSEG = 512
NROW = 320000
NCOL = 128
NC = 2
NS = 16
CHALF = NCOL // NC
RPT = NROW // NS
BLK = 125
NB = RPT // BLK
SEG_PT = NSEG // NS
CW = 16


def _body(x_hbm, ids_hbm, out_hbm,
          ids_v, xb0, xb1, xb2, xb3, ones_v, zb, zc, sbuf, tbuf, cbuf, ctbuf,
          obuf, tab_e, tab_o, cnt_e, cnt_o,
          sl0, sl1, sl2, sl3, ste, sto, sce, sco):
    c = lax.axis_index("c")
    s = lax.axis_index("s")
    col0 = c * CHALF
    row0 = s * RPT
    seg0 = s * SEG_PT

    xbs = (xb0, xb1, xb2, xb3)
    sls = (sl0, sl1, sl2, sl3)
    tabs = (tab_e, tab_o)
    cnts = (cnt_e, cnt_o)
    sts = (ste, sto)
    scs = (sce, sco)

    pltpu.sync_copy(ids_hbm.at[s], ids_v)

    one16 = jnp.ones((16,), jnp.float32)
    zero16 = jnp.zeros((16,), jnp.float32)
    for r in range(BLK):
        ones_v[r, :] = one16
    for r in range(SEG_PT):
        zc[r, :] = zero16
    for r in range(SEG_PT):
        for j in range(CHALF // 16):
            zb[r, pl.ds(j * 16, 16)] = zero16

    pltpu.sync_copy(zb, tab_e.at[pl.ds(seg0, SEG_PT), :])
    pltpu.sync_copy(zb, tab_o.at[pl.ds(seg0, SEG_PT), :])
    pltpu.sync_copy(zc, cnt_e.at[pl.ds(seg0, SEG_PT), :])
    pltpu.sync_copy(zc, cnt_o.at[pl.ds(seg0, SEG_PT), :])
    plsc.subcore_barrier()

    def start_load(b_idx, k):
        pltpu.async_copy(
            x_hbm.at[pl.ds(row0 + b_idx * BLK, BLK), pl.ds(col0, CHALF)],
            xbs[k], sls[k])

    def wait_load(k):
        pltpu.make_async_copy(
            x_hbm.at[pl.ds(0, BLK), pl.ds(0, CHALF)], xbs[k], sls[k]).wait()

    def start_scatter(b_idx, k):
        p = k % 2
        idx = ids_v.at[b_idx]
        pltpu.async_copy(ones_v, cnts[p].at[idx], scs[p], add=True)
        pltpu.async_copy(xbs[k], tabs[p].at[idx], sts[p], add=True)

    def wait_scatter(p):
        pltpu.make_async_copy(ones_v, cnts[p].at[ids_v.at[0]], scs[p]).wait()
        pltpu.make_async_copy(xbs[p], tabs[p].at[ids_v.at[0]], sts[p]).wait()

    start_load(0, 0)
    start_load(1, 1)
    start_load(2, 2)
    wait_load(0)
    start_scatter(0, 0)
    start_load(3, 3)
    wait_load(1)
    start_scatter(1, 1)
    wait_scatter(0)
    start_load(4, 0)
    wait_load(2)
    start_scatter(2, 2)
    wait_scatter(1)
    start_load(5, 1)
    wait_load(3)
    start_scatter(3, 3)

    def quad(i, carry):
        g = i * 4
        for k in range(4):
            wait_scatter(k % 2)
            start_load(g + k + 2, (k + 2) % 4)
            wait_load(k)
            start_scatter(g + k, k)
        return carry

    lax.fori_loop(1, NB // 4 - 1, quad, 0)

    for k in range(4):
        g = NB - 4 + k
        wait_scatter(k % 2)
        if k < 2:
            start_load(NB - 2 + k, k + 2)
        wait_load(k)
        start_scatter(g, k)
    wait_scatter(0)
    wait_scatter(1)

    plsc.subcore_barrier()

    pltpu.sync_copy(tab_e.at[pl.ds(seg0, SEG_PT), :], sbuf)
    pltpu.sync_copy(tab_o.at[pl.ds(seg0, SEG_PT), :], tbuf)
    pltpu.sync_copy(cnt_e.at[pl.ds(seg0, SEG_PT), :], cbuf)
    pltpu.sync_copy(cnt_o.at[pl.ds(seg0, SEG_PT), :], ctbuf)
    for r in range(SEG_PT):
        cv = cbuf[r, :] + ctbuf[r, :]
        rv = 1.0 / jnp.maximum(cv, 1.0)
        for j in range(CHALF // 16):
            sl = pl.ds(j * 16, 16)
            obuf[r, sl] = (sbuf[r, sl] + tbuf[r, sl]) * rv
    pltpu.sync_copy(obuf, out_hbm.at[pl.ds(seg0, SEG_PT), pl.ds(col0, CHALF)])


_pool = functools.partial(
    pl.kernel,
    out_type=jax.ShapeDtypeStruct((NSEG, NCOL), jnp.float32),
    mesh=plsc.VectorSubcoreMesh(core_axis_name="c", subcore_axis_name="s",
                                num_cores=NC, num_subcores=NS),
    compiler_params=pltpu.CompilerParams(use_tc_tiling_on_sc=False,
                                         needs_layout_passes=False),
    scratch_types=[
        pltpu.VMEM((NB, BLK), jnp.int32),
        pltpu.VMEM((BLK, CHALF), jnp.float32),
        pltpu.VMEM((BLK, CHALF), jnp.float32),
        pltpu.VMEM((BLK, CHALF), jnp.float32),
        pltpu.VMEM((BLK, CHALF), jnp.float32),
        pltpu.VMEM((BLK, CW), jnp.float32),
        pltpu.VMEM((SEG_PT, CHALF), jnp.float32),
        pltpu.VMEM((SEG_PT, CW), jnp.float32),
        pltpu.VMEM((SEG_PT, CHALF), jnp.float32),
        pltpu.VMEM((SEG_PT, CHALF), jnp.float32),
        pltpu.VMEM((SEG_PT, CW), jnp.float32),
        pltpu.VMEM((SEG_PT, CW), jnp.float32),
        pltpu.VMEM((SEG_PT, CHALF), jnp.float32),
        pltpu.VMEM_SHARED((NSEG, CHALF), jnp.float32),
        pltpu.VMEM_SHARED((NSEG, CHALF), jnp.float32),
        pltpu.VMEM_SHARED((NSEG, CW), jnp.float32),
        pltpu.VMEM_SHARED((NSEG, CW), jnp.float32),
        pltpu.SemaphoreType.DMA,
        pltpu.SemaphoreType.DMA,
        pltpu.SemaphoreType.DMA,
        pltpu.SemaphoreType.DMA,
        pltpu.SemaphoreType.DMA,
        pltpu.SemaphoreType.DMA,
        pltpu.SemaphoreType.DMA,
        pltpu.SemaphoreType.DMA,
    ],
)(_body)


@jax.jit
def kernel(x, batch):
    ids = batch.astype(jnp.int32).reshape(NS, NB, BLK)
    return _pool(x, ids)
